# Initial kernel scaffold; baseline (speedup 1.0000x reference)
import jax
import jax.numpy as jnp
from jax.experimental import pallas as pl


def kernel(table, indices, anchor_index):
    raise NotImplementedError("write your pallas kernel here")


# speedup vs baseline: 1.7082x; 1.7082x over previous
import functools

import numpy as np
import jax
import jax.numpy as jnp
from jax import lax
from jax.experimental import pallas as pl
from jax.experimental.pallas import tpu as pltpu
from jax.experimental.pallas import tpu_sc as plsc

D = 64
B = 4096
L = 200
W = 2
K = 2 * W + 1

NC = 2
NS = 16
NW = NC * NS
BPW = B // NW
NLANES = 16
NGROUP = (BPW * K) // NLANES


def _sc_window_gather(table, idx_flat, anchor_index):
  mesh = plsc.VectorSubcoreMesh(core_axis_name="c", subcore_axis_name="s",
                                num_cores=NC, num_subcores=NS)

  r_all = np.arange(NGROUP * NLANES, dtype=np.int32)
  b_map = jnp.asarray(r_all // K)
  off_map = jnp.asarray(r_all % K - W)

  @functools.partial(
      pl.kernel,
      mesh=mesh,
      compiler_params=pltpu.CompilerParams(needs_layout_passes=False,
                                           use_tc_tiling_on_sc=False),
      out_type=jax.ShapeDtypeStruct((B * K, D), jnp.float32),
      scratch_types=[
          pltpu.VMEM((BPW * L + BPW,), jnp.int32),
          pltpu.VMEM((K * BPW,), jnp.int32),
          pltpu.VMEM((K * BPW,), jnp.int32),
          [pltpu.VMEM((BPW,), jnp.int32) for _ in range(K)],
          pltpu.VMEM((K * BPW, D), jnp.float32),
          pltpu.SemaphoreType.DMA,
      ],
  )
  def k(table_hbm, idx_hbm, anc_hbm, bmap_hbm, omap_hbm, out_hbm,
        idx_v, bmap_v, omap_v, toks, rows_v, sem):
    wid = lax.axis_index("s") * NC + lax.axis_index("c")
    base = wid * BPW
    anc_off = BPW * L
    pltpu.sync_copy(idx_hbm.at[pl.ds(base * L, BPW * L)],
                    idx_v.at[pl.ds(0, BPW * L)])
    pltpu.sync_copy(anc_hbm.at[pl.ds(base, BPW)],
                    idx_v.at[pl.ds(anc_off, BPW)])
    pltpu.sync_copy(bmap_hbm, bmap_v)
    pltpu.sync_copy(omap_hbm, omap_v)

    lanes = lax.iota(jnp.int32, NLANES)
    for g in range(NGROUP):
      b_loc = bmap_v[pl.ds(g * NLANES, NLANES)]
      off = omap_v[pl.ds(g * NLANES, NLANES)]
      a = plsc.load_gather(idx_v, [anc_off + b_loc])
      pos = jnp.clip(a + off, 0, L - 1)
      tok = plsc.load_gather(idx_v, [b_loc * L + pos])
      toks[g * NLANES // BPW][pl.ds((g * NLANES) % BPW, NLANES)] = tok

    copies = [
        pltpu.async_copy(table_hbm.at[toks[kk]],
                         rows_v.at[pl.ds(kk * BPW, BPW), :], sem)
        for kk in range(K)
    ]
    for c in copies:
      c.wait()

    zero16 = jnp.zeros((NLANES,), jnp.float32)
    for g in range(NGROUP):
      b_loc = bmap_v[pl.ds(g * NLANES, NLANES)]
      off = omap_v[pl.ds(g * NLANES, NLANES)]
      a = plsc.load_gather(idx_v, [anc_off + b_loc])
      pos = a + off
      inv = ((pos < 0) | (pos >= L)).astype(jnp.int32)
      n_inv = jnp.sum(inv)

      @pl.when(n_inv > 0)
      def _fix(g=g, inv=inv):
        def body(lane, carry):
          bad = jnp.sum(jnp.where(lanes == lane, inv, 0)) > 0

          @pl.when(bad)
          def _zero():
            rr = g * NLANES + lane
            for c4 in range(D // NLANES):
              rows_v[rr, pl.ds(c4 * NLANES, NLANES)] = zero16
          return carry
        lax.fori_loop(0, NLANES, body, 0)

    pltpu.sync_copy(rows_v, out_hbm.at[pl.ds(base * K, BPW * K), :])

  return k(table, idx_flat, anchor_index, b_map, off_map)


def kernel(table, indices, anchor_index):
  out = _sc_window_gather(table, indices.reshape(-1), anchor_index)
  return out.reshape(B, K * D)
